# baseline (speedup 1.0000x reference)
import functools

import numpy as np
import jax
import jax.numpy as jnp
from jax.experimental import pallas as pl
from jax.experimental.pallas import tpu as pltpu

_EPS = 1e-6


def _gem_body(p_ref, x_ref, s_ref, o_ref):
    p = p_ref[0]
    inv_p = 1.0 / p

    xc = jnp.maximum(x_ref[...], _EPS)
    zp = jnp.exp2(p * jnp.log2(xc))

    pooled = jnp.dot(zp, s_ref[...], preferred_element_type=jnp.float32)

    o_ref[...] = jnp.exp2(inv_p * jnp.log2(pooled)).astype(o_ref.dtype)


def _segment_mean_matrix(hw, bins):
    seg = hw // bins
    m = np.zeros((hw, bins), dtype=np.float32)
    m[np.arange(hw), np.arange(hw) // seg] = 1.0 / seg
    return jnp.asarray(m)


def _pick_tile_rows(rows, hw, budget_bytes=2 * 1024 * 1024):
    max_tile = max(8, budget_bytes // (hw * 4))
    if rows <= max_tile:
        return rows
    t = (max_tile // 8) * 8
    while t >= 8:
        if rows % t == 0:
            return t
        t -= 8
    return rows


@functools.partial(jax.jit, static_argnames=("bins",))
def _gem_hpp_2d(x2, p_scalar, bins):
    rows, hw = x2.shape
    s = _segment_mean_matrix(hw, bins)
    tile_r = _pick_tile_rows(rows, hw)

    return pl.pallas_call(
        _gem_body,
        out_shape=jax.ShapeDtypeStruct((rows, bins), x2.dtype),
        grid=(rows // tile_r,),
        in_specs=[
            pl.BlockSpec(memory_space=pltpu.MemorySpace.SMEM),
            pl.BlockSpec((tile_r, hw), lambda i: (i, 0)),
            pl.BlockSpec((hw, bins), lambda i: (0, 0)),
        ],
        out_specs=pl.BlockSpec((tile_r, bins), lambda i: (i, 0)),
        compiler_params=pltpu.CompilerParams(
            dimension_semantics=("parallel",)),
    )(p_scalar, x2, s)


def kernel(x, p_scalar):
    n, c, h, w = x.shape
    bins = 64
    x2 = x.reshape(n * c, h * w)
    out = _gem_hpp_2d(x2, p_scalar, bins)
    return out.reshape(n, c, bins)


# speedup vs baseline: 5.3949x; 5.3949x over previous
import functools

import numpy as np
import jax
import jax.numpy as jnp
from jax.experimental import pallas as pl
from jax.experimental.pallas import tpu as pltpu

_EPS = 1e-6


def _gem_body(p_ref, x_ref, st_ref, o_ref):
    p = p_ref[0]
    inv_p = 1.0 / p

    xc = jnp.maximum(x_ref[0], _EPS)
    zp = jnp.exp2(p * jnp.log2(xc))

    pooled = jnp.dot(st_ref[...], zp, preferred_element_type=jnp.float32)

    o_ref[0] = jnp.exp2(inv_p * jnp.log2(pooled)).astype(o_ref.dtype)


def _segment_mean_matrix_t(hw, bins):
    seg = hw // bins
    m = np.zeros((bins, hw), dtype=np.float32)
    m[np.arange(hw) // seg, np.arange(hw)] = 1.0 / seg
    return jnp.asarray(m)


@functools.partial(jax.jit, static_argnames=("bins",))
def _gem_hpp_nhwc(xt, p_scalar, bins):
    n, hw, c = xt.shape
    st = _segment_mean_matrix_t(hw, bins)

    return pl.pallas_call(
        _gem_body,
        out_shape=jax.ShapeDtypeStruct((n, bins, c), xt.dtype),
        grid=(n,),
        in_specs=[
            pl.BlockSpec(memory_space=pltpu.MemorySpace.SMEM),
            pl.BlockSpec((1, hw, c), lambda i: (i, 0, 0)),
            pl.BlockSpec((bins, hw), lambda i: (0, 0)),
        ],
        out_specs=pl.BlockSpec((1, bins, c), lambda i: (i, 0, 0)),
        compiler_params=pltpu.CompilerParams(
            dimension_semantics=("parallel",)),
    )(p_scalar, xt, st)


def kernel(x, p_scalar):
    n, c, h, w = x.shape
    bins = 64
    xt = x.transpose(0, 2, 3, 1).reshape(n, h * w, c)
    out = _gem_hpp_nhwc(xt, p_scalar, bins)
    return out.transpose(0, 2, 1)


# speedup vs baseline: 6.9088x; 1.2806x over previous
import functools

import numpy as np
import jax
import jax.numpy as jnp
from jax.experimental import pallas as pl
from jax.experimental.pallas import tpu as pltpu

_EPS = 1e-6


def _gem_body(p_ref, x_ref, st_ref, o_ref):
    p = p_ref[0]
    inv_p = 1.0 / p

    for b in range(x_ref.shape[0]):
        xc = jnp.maximum(x_ref[b], _EPS)
        zp = jnp.exp2(p * jnp.log2(xc))

        pooled = jnp.dot(st_ref[...], zp, preferred_element_type=jnp.float32)

        o_ref[b] = jnp.exp2(inv_p * jnp.log2(pooled)).astype(o_ref.dtype)


def _segment_mean_matrix_t(hw, bins):
    seg = hw // bins
    m = np.zeros((bins, hw), dtype=np.float32)
    m[np.arange(hw) // seg, np.arange(hw)] = 1.0 / seg
    return jnp.asarray(m)


@functools.partial(jax.jit, static_argnames=("bins",))
def _gem_hpp_nhwc(xt, p_scalar, bins):
    n, hw, c = xt.shape
    st = _segment_mean_matrix_t(hw, bins)

    tile_n = 2 if n % 2 == 0 else 1

    return pl.pallas_call(
        _gem_body,
        out_shape=jax.ShapeDtypeStruct((n, bins, c), xt.dtype),
        grid=(n // tile_n,),
        in_specs=[
            pl.BlockSpec(memory_space=pltpu.MemorySpace.SMEM),
            pl.BlockSpec((tile_n, hw, c), lambda i: (i, 0, 0)),
            pl.BlockSpec((bins, hw), lambda i: (0, 0)),
        ],
        out_specs=pl.BlockSpec((tile_n, bins, c), lambda i: (i, 0, 0)),
        compiler_params=pltpu.CompilerParams(
            dimension_semantics=("parallel",)),
    )(p_scalar, xt, st)


def kernel(x, p_scalar):
    n, c, h, w = x.shape
    bins = 64
    xt = x.transpose(0, 2, 3, 1).reshape(n, h * w, c)
    out = _gem_hpp_nhwc(xt, p_scalar, bins)
    return out.transpose(0, 2, 1)


# speedup vs baseline: 7.7769x; 1.1257x over previous
import functools

import numpy as np
import jax
import jax.numpy as jnp
from jax.experimental import pallas as pl
from jax.experimental.pallas import tpu as pltpu

_EPS = 1e-6


def _gem_body(p_ref, x_ref, st_ref, o_ref):
    p = p_ref[0]
    inv_p = 1.0 / p

    for b in range(x_ref.shape[0]):
        xc = jnp.maximum(x_ref[b], _EPS)
        zp = jnp.exp2(p * jnp.log2(xc))

        pooled = jnp.dot(st_ref[...], zp, preferred_element_type=jnp.float32)

        o_ref[b] = jnp.exp2(inv_p * jnp.log2(pooled)).astype(o_ref.dtype)


def _segment_mean_matrix_t(hw, bins):
    seg = hw // bins
    m = np.zeros((bins, hw), dtype=np.float32)
    m[np.arange(hw) // seg, np.arange(hw)] = 1.0 / seg
    return jnp.asarray(m)


@functools.partial(jax.jit, static_argnames=("bins",))
def _gem_hpp_nhwc(xt, p_scalar, bins):
    n, hw, c = xt.shape
    st = _segment_mean_matrix_t(hw, bins)

    tile_n = 4 if n % 4 == 0 else (2 if n % 2 == 0 else 1)

    return pl.pallas_call(
        _gem_body,
        out_shape=jax.ShapeDtypeStruct((n, bins, c), xt.dtype),
        grid=(n // tile_n,),
        in_specs=[
            pl.BlockSpec(memory_space=pltpu.MemorySpace.SMEM),
            pl.BlockSpec((tile_n, hw, c), lambda i: (i, 0, 0)),
            pl.BlockSpec((bins, hw), lambda i: (0, 0)),
        ],
        out_specs=pl.BlockSpec((tile_n, bins, c), lambda i: (i, 0, 0)),
        compiler_params=pltpu.CompilerParams(
            dimension_semantics=("parallel",)),
    )(p_scalar, xt, st)


def kernel(x, p_scalar):
    n, c, h, w = x.shape
    bins = 64
    xt = x.transpose(0, 2, 3, 1).reshape(n, h * w, c)
    out = _gem_hpp_nhwc(xt, p_scalar, bins)
    return out.transpose(0, 2, 1)


# speedup vs baseline: 8.1652x; 1.0499x over previous
import functools

import numpy as np
import jax
import jax.numpy as jnp
from jax.experimental import pallas as pl
from jax.experimental.pallas import tpu as pltpu

_EPS = 1e-6
_NBUF = 4
_TILE_N = 2


def _gem_body(p_ref, x_hbm, st_ref, o_ref, x_buf, in_sem):
    p = p_ref[0]
    inv_p = 1.0 / p
    n_chunks = x_hbm.shape[0] // _TILE_N

    def start_in(slot, chunk):
        pltpu.make_async_copy(
            x_hbm.at[pl.ds(chunk * _TILE_N, _TILE_N)],
            x_buf.at[slot], in_sem.at[slot]).start()

    def wait_in(slot):
        pltpu.make_async_copy(
            x_hbm.at[pl.ds(0, _TILE_N)],
            x_buf.at[slot], in_sem.at[slot]).wait()

    for c0 in range(min(_NBUF, n_chunks)):
        start_in(c0, c0)

    def body(i, _):
        s = jax.lax.rem(i, _NBUF)
        wait_in(s)
        for b in range(_TILE_N):
            xc = jnp.maximum(x_buf[s, b], _EPS)
            zp = jnp.exp2(p * jnp.log2(xc))
            pooled = jnp.dot(st_ref[...], zp,
                             preferred_element_type=jnp.float32)
            o_ref[i * _TILE_N + b] = jnp.exp2(inv_p * jnp.log2(pooled))

        @pl.when(i + _NBUF < n_chunks)
        def _():
            start_in(s, i + _NBUF)
        return 0

    jax.lax.fori_loop(0, n_chunks, body, 0)


def _segment_mean_matrix_t(hw, bins):
    seg = hw // bins
    m = np.zeros((bins, hw), dtype=np.float32)
    m[np.arange(hw) // seg, np.arange(hw)] = 1.0 / seg
    return jnp.asarray(m)


@functools.partial(jax.jit, static_argnames=("bins",))
def _gem_hpp_nhwc(xt, p_scalar, bins):
    n, hw, c = xt.shape
    st = _segment_mean_matrix_t(hw, bins)

    return pl.pallas_call(
        _gem_body,
        out_shape=jax.ShapeDtypeStruct((n, bins, c), xt.dtype),
        in_specs=[
            pl.BlockSpec(memory_space=pltpu.MemorySpace.SMEM),
            pl.BlockSpec(memory_space=pltpu.MemorySpace.HBM),
            pl.BlockSpec(memory_space=pltpu.MemorySpace.VMEM),
        ],
        out_specs=pl.BlockSpec(memory_space=pltpu.MemorySpace.VMEM),
        scratch_shapes=[
            pltpu.VMEM((_NBUF, _TILE_N, hw, c), jnp.float32),
            pltpu.SemaphoreType.DMA((_NBUF,)),
        ],
    )(p_scalar, xt, st)


def kernel(x, p_scalar):
    n, c, h, w = x.shape
    bins = 64
    xt = x.transpose(0, 2, 3, 1).reshape(n, h * w, c)
    out = _gem_hpp_nhwc(xt, p_scalar, bins)
    return out.transpose(0, 2, 1)
